# Initial kernel scaffold; baseline (speedup 1.0000x reference)
import jax
import jax.numpy as jnp
from jax.experimental import pallas as pl


def kernel(x, edge_index, assign_nodes, assign_set_ids, W0_root, W0_rel, b0, W1_root, W1_rel, b1, W2_root, W2_rel, b2, fc1_w, fc1_b, fc2_w, fc2_b, fc3_w, fc3_b):
    raise NotImplementedError("write your pallas kernel here")


# speedup vs baseline: 4.9984x; 4.9984x over previous
import functools

import jax
import jax.numpy as jnp
from jax import lax
from jax.experimental import pallas as pl
from jax.experimental.pallas import tpu as pltpu
from jax.experimental.pallas import tpu_sc as plsc

N = 10000
E = 320000
D = 128
W = 64
A = 40000
S = 5000

NC = 2
NT = 16
CH = 128

NPAD = 10112
SPAD = 5120

ECHUNKS_PER_TILE = 79
ACHUNKS_PER_TILE = 10

BM = 1000


def _elu(v):
    return jnp.where(v > 0, v, jnp.exp(jnp.minimum(v, 0.0)) - 1.0)


def _make_sc_segment_sum(npad, chunks_per_tile, width):
    mesh = plsc.VectorSubcoreMesh(core_axis_name="c", subcore_axis_name="s")
    zr = npad // NT

    @functools.partial(
        pl.kernel,
        out_type=jax.ShapeDtypeStruct((NC * npad, width), jnp.float32),
        mesh=mesh,
        scratch_types=[
            pltpu.VMEM((2, CH), jnp.int32),
            pltpu.VMEM((CH, width), jnp.float32),
            pltpu.VMEM_SHARED((npad, width), jnp.float32),
            pltpu.SemaphoreType.DMA,
        ],
        compiler_params=pltpu.CompilerParams(use_tc_tiling_on_sc=False),
    )
    def seg(table_hbm, idx_hbm, out_hbm, idxb, rows, acc, sem):
        cid = lax.axis_index("c")
        sid = lax.axis_index("s")

        z16 = jnp.zeros((16,), jnp.float32)

        def zbody(i, carry):
            for k in range(width // 16):
                rows[i, pl.ds(k * 16, 16)] = z16
            return carry

        lax.fori_loop(0, CH, zbody, 0)
        zbase = pl.multiple_of(sid * zr, 8)
        nfull, rem = zr // CH, zr % CH
        for q in range(nfull):
            pltpu.sync_copy(rows, acc.at[pl.ds(zbase + q * CH, CH)])
        if rem:
            pltpu.sync_copy(rows.at[pl.ds(0, rem)],
                            acc.at[pl.ds(zbase + nfull * CH, rem)])
        plsc.subcore_barrier()

        c0 = (cid * NT + sid) * chunks_per_tile

        def body(j, carry):
            pltpu.sync_copy(idx_hbm.at[c0 + j], idxb)
            pltpu.async_copy(table_hbm.at[idxb.at[0]], rows, sem).wait()
            pltpu.sync_copy(rows, acc.at[idxb.at[1]], add=True)
            return carry

        lax.fori_loop(0, chunks_per_tile, body, 0)
        plsc.subcore_barrier()

        pltpu.sync_copy(acc.at[pl.ds(zbase, zr)],
                        out_hbm.at[pl.ds(pl.multiple_of(cid * npad + zbase, 8), zr)])

    return seg


_seg_edge_x = _make_sc_segment_sum(NPAD, ECHUNKS_PER_TILE, D)
_seg_edge_h = _make_sc_segment_sum(NPAD, ECHUNKS_PER_TILE, W)
_seg_pool = _make_sc_segment_sum(SPAD, ACHUNKS_PER_TILE, W)


def _pack_idx(gidx, sidx, total, dump_row):
    n = gidx.shape[0]
    pad = total - n
    g = jnp.concatenate([gidx.astype(jnp.int32),
                         jnp.zeros((pad,), jnp.int32)])
    s = jnp.concatenate([sidx.astype(jnp.int32),
                         jnp.full((pad,), dump_row, jnp.int32)])
    return jnp.stack([g.reshape(-1, CH), s.reshape(-1, CH)], axis=1)


def _layer_tc(h, agg0, agg1, w_root, w_rel, b):
    din = h.shape[1]

    def body(h_ref, a0_ref, a1_ref, wr_ref, wl_ref, b_ref, o_ref):
        agg = a0_ref[...] + a1_ref[...]
        o_ref[...] = _elu(
            jnp.dot(h_ref[...], wr_ref[...], preferred_element_type=jnp.float32)
            + jnp.dot(agg, wl_ref[...], preferred_element_type=jnp.float32)
            + b_ref[...])

    return pl.pallas_call(
        body,
        grid=(N // BM,),
        in_specs=[
            pl.BlockSpec((BM, din), lambda i: (i, 0)),
            pl.BlockSpec((BM, din), lambda i: (i, 0)),
            pl.BlockSpec((BM, din), lambda i: (i, 0)),
            pl.BlockSpec((din, W), lambda i: (0, 0)),
            pl.BlockSpec((din, W), lambda i: (0, 0)),
            pl.BlockSpec((1, W), lambda i: (0, 0)),
        ],
        out_specs=pl.BlockSpec((BM, W), lambda i: (i, 0)),
        out_shape=jax.ShapeDtypeStruct((N, W), jnp.float32),
    )(h, agg0, agg1, w_root, w_rel, b.reshape(1, W))


def _head(p0, p1, fc1_w, fc1_b, fc2_w, fc2_b, fc3_w, fc3_b):
    def body(p0_ref, p1_ref, w1_ref, b1_ref, w2_ref, b2_ref, w3_ref, b3_ref, o_ref):
        p = p0_ref[...] + p1_ref[...]
        h = _elu(jnp.dot(p, w1_ref[...], preferred_element_type=jnp.float32) + b1_ref[...])
        h = _elu(jnp.dot(h, w2_ref[...], preferred_element_type=jnp.float32) + b2_ref[...])
        lg = jnp.dot(h, w3_ref[...], preferred_element_type=jnp.float32) + b3_ref[...]
        m = jnp.max(lg, axis=1, keepdims=True)
        e = jnp.exp(lg - m)
        o_ref[...] = (lg - m) - jnp.log(jnp.sum(e, axis=1, keepdims=True))

    return pl.pallas_call(
        body,
        grid=(S // BM,),
        in_specs=[
            pl.BlockSpec((BM, W), lambda i: (i, 0)),
            pl.BlockSpec((BM, W), lambda i: (i, 0)),
            pl.BlockSpec((W, W), lambda i: (0, 0)),
            pl.BlockSpec((1, W), lambda i: (0, 0)),
            pl.BlockSpec((W, 32), lambda i: (0, 0)),
            pl.BlockSpec((1, 32), lambda i: (0, 0)),
            pl.BlockSpec((32, 2), lambda i: (0, 0)),
            pl.BlockSpec((1, 2), lambda i: (0, 0)),
        ],
        out_specs=pl.BlockSpec((BM, 2), lambda i: (i, 0)),
        out_shape=jax.ShapeDtypeStruct((S, 2), jnp.float32),
    )(p0, p1, fc1_w, fc1_b.reshape(1, W), fc2_w, fc2_b.reshape(1, 32),
      fc3_w, fc3_b.reshape(1, 2))


def kernel(x, edge_index, assign_nodes, assign_set_ids,
           W0_root, W0_rel, b0,
           W1_root, W1_rel, b1,
           W2_root, W2_rel, b2,
           fc1_w, fc1_b, fc2_w, fc2_b, fc3_w, fc3_b):
    src = edge_index[0].astype(jnp.int32)
    dst = edge_index[1].astype(jnp.int32)
    eidx = _pack_idx(src, dst, NC * NT * ECHUNKS_PER_TILE * CH, NPAD - 1)
    aidx = _pack_idx(assign_set_ids, assign_nodes,
                     NC * NT * ACHUNKS_PER_TILE * CH, SPAD - 1)

    agg = _seg_edge_x(x, eidx)
    h1 = _layer_tc(x, agg[:N], agg[NPAD:NPAD + N], W0_root, W0_rel, b0)
    agg = _seg_edge_h(h1, eidx)
    h2 = _layer_tc(h1, agg[:N], agg[NPAD:NPAD + N], W1_root, W1_rel, b1)
    agg = _seg_edge_h(h2, eidx)
    h3 = _layer_tc(h2, agg[:N], agg[NPAD:NPAD + N], W2_root, W2_rel, b2)

    pooled = _seg_pool(h3, aidx)
    return _head(pooled[:S], pooled[SPAD:SPAD + S],
                 fc1_w, fc1_b, fc2_w, fc2_b, fc3_w, fc3_b)
